# baseline (speedup 1.0000x reference)
import math

import jax
import jax.numpy as jnp
from jax import lax
from jax.experimental import pallas as pl
from jax.experimental.pallas import tpu as pltpu
from jax.experimental.pallas import tpu_sc as plsc

_NUM_LEVELS = 16
_F = 2
_T = 2 ** 19
_MASK = _T - 1
_LW = _T * _F
_BASE_RES = 16
_FINEST_RES = 2048
_GROWTH = math.exp(
    (math.log(_FINEST_RES) - math.log(_BASE_RES)) / (_NUM_LEVELS - 1))
_RES = [max(1, int(round(_BASE_RES * _GROWTH ** l)))
        for l in range(_NUM_LEVELS)]
_N = 262144
_NC, _NS = 2, 16
_NW = _NC * _NS
_PW = _N // _NW
_C = 128
_NCHUNK = _PW // _C
_G = _C // 16
_PA = 73856093
_PB = 19349663
_PC = 83492791


def _body(xt_hbm, tab_hbm, resb_hbm, scr_hbm, out_hbm, sh,
          x0, x1, idxE0, idxE1, idxO0, idxO1, fE0, fE1, fO0, fO1,
          res_v, resb_v, tr_v, out_v, sem0, sem1):
    cid = lax.convert_element_type(lax.axis_index("c"), jnp.int32)
    sid = lax.convert_element_type(lax.axis_index("s"), jnp.int32)
    wid32 = sid * jnp.int32(_NC) + cid
    lane = lax.iota(jnp.int32, 16)
    x_b = (x0, x1)
    idxE_b = (idxE0, idxE1)
    idxO_b = (idxO0, idxO1)
    fE_b = (fE0, fE1)
    fO_b = (fO0, fO1)
    sem_b = (sem0, sem1)
    wbase = wid32 * jnp.int32(_PW)
    stage_per = _LW // _NS

    def pass1(p, pbase, res):
        x_v, idxE, idxO = x_b[p], idxE_b[p], idxO_b[p]
        for d in range(3):
            pltpu.sync_copy(
                xt_hbm.at[pl.ds(jnp.int32(d * _N) + pbase, _C)],
                x_v.at[jnp.int32(d)])

        def grp(g, c2):
            s = g * jnp.int32(16)
            xs0 = x_v[jnp.int32(0), pl.ds(s, 16)] * res
            xs1 = x_v[jnp.int32(1), pl.ds(s, 16)] * res
            xs2 = x_v[jnp.int32(2), pl.ds(s, 16)] * res
            i0 = xs0.astype(jnp.int32)
            i1 = xs1.astype(jnp.int32)
            i2 = xs2.astype(jnp.int32)
            a0 = i0 * jnp.int32(_PA)
            a1 = a0 + jnp.int32(_PA)
            b0 = i1 * jnp.int32(_PB)
            b1 = b0 + jnp.int32(_PB)
            c0 = i2 * jnp.int32(_PC)
            c1 = c0 + jnp.int32(_PC)
            corner = 0
            for aa in (a0, a1):
                for bb in (b0, b1):
                    for cc in (c0, c1):
                        h = (aa ^ bb ^ cc) & jnp.int32(_MASK)
                        e = lax.shift_left(h, jnp.int32(1))
                        idxE[jnp.int32(corner), pl.ds(s, 16)] = e
                        idxO[jnp.int32(corner), pl.ds(s, 16)] = \
                            e + jnp.int32(1)
                        corner += 1
            return c2

        lax.fori_loop(jnp.int32(0), jnp.int32(_G), grp, jnp.int32(0))
        for c in range(8):
            pltpu.async_copy(sh.at[idxE.at[jnp.int32(c)]],
                             fE_b[p].at[jnp.int32(c)], sem_b[p])
            pltpu.async_copy(sh.at[idxO.at[jnp.int32(c)]],
                             fO_b[p].at[jnp.int32(c)], sem_b[p])

    def drain(p):
        dummy = tab_hbm.at[pl.ds(0, _C)]
        for c in range(8):
            pltpu.make_async_copy(dummy, fE_b[p].at[jnp.int32(c)],
                                  sem_b[p]).wait()
            pltpu.make_async_copy(dummy, fO_b[p].at[jnp.int32(c)],
                                  sem_b[p]).wait()

    def combine(p, pbase, res, l):
        x_v, fE, fO = x_b[p], fE_b[p], fO_b[p]

        def grp(g, c2):
            s = g * jnp.int32(16)
            xs0 = x_v[jnp.int32(0), pl.ds(s, 16)] * res
            xs1 = x_v[jnp.int32(1), pl.ds(s, 16)] * res
            xs2 = x_v[jnp.int32(2), pl.ds(s, 16)] * res
            w0 = xs0 - xs0.astype(jnp.int32).astype(jnp.float32)
            w1 = xs1 - xs1.astype(jnp.int32).astype(jnp.float32)
            w2 = xs2 - xs2.astype(jnp.int32).astype(jnp.float32)
            u0 = 1.0 - w0
            u1 = 1.0 - w1
            u2 = 1.0 - w2
            p00 = u0 * u1
            p01 = u0 * w1
            p10 = w0 * u1
            p11 = w0 * w1
            wts = (p00 * u2, p00 * w2, p01 * u2, p01 * w2,
                   p10 * u2, p10 * w2, p11 * u2, p11 * w2)
            acc0 = None
            acc1 = None
            for c in range(8):
                f0 = fE[jnp.int32(c), pl.ds(s, 16)]
                f1 = fO[jnp.int32(c), pl.ds(s, 16)]
                t0 = wts[c] * f0
                t1 = wts[c] * f1
                acc0 = t0 if acc0 is None else acc0 + t0
                acc1 = t1 if acc1 is None else acc1 + t1
            pat = lax.shift_left(s + lane, jnp.int32(1))
            plsc.store_scatter(res_v, [pat], acc0)
            plsc.store_scatter(res_v, [pat + jnp.int32(1)], acc1)
            return c2

        lax.fori_loop(jnp.int32(0), jnp.int32(_G), grp, jnp.int32(0))
        pltpu.sync_copy(
            res_v,
            scr_hbm.at[pl.ds(l * jnp.int32(2 * _N) + pbase * jnp.int32(2),
                             2 * _C)])

    pltpu.sync_copy(resb_hbm, resb_v)

    def level_body(l, carry0):
        res = resb_v[pl.ds(l * jnp.int32(16), 16)]
        plsc.subcore_barrier()
        pltpu.sync_copy(
            tab_hbm.at[pl.ds(l * jnp.int32(_LW)
                             + sid * jnp.int32(stage_per), stage_per)],
            sh.at[pl.ds(sid * jnp.int32(stage_per), stage_per)])
        plsc.subcore_barrier()

        pass1(0, wbase, res)

        def pair_body(j, carry):
            base0 = wbase + j * jnp.int32(2 * _C)
            pass1(1, base0 + jnp.int32(_C), res)
            drain(0)
            combine(0, base0, res, l)
            @pl.when(j < jnp.int32(_NCHUNK // 2 - 1))
            def _():
                pass1(0, base0 + jnp.int32(2 * _C), res)
            drain(1)
            combine(1, base0 + jnp.int32(_C), res, l)
            return carry

        lax.fori_loop(jnp.int32(0), jnp.int32(_NCHUNK // 2), pair_body,
                      jnp.int32(0))
        return carry0

    lax.fori_loop(jnp.int32(0), jnp.int32(_NUM_LEVELS), level_body,
                  jnp.int32(0))

    patL = lax.shift_right_logical(lane, jnp.int32(1)) * jnp.int32(32) \
        + (lane & jnp.int32(1))

    def tr_chunk(ci, carry):
        pbase = wbase + ci * jnp.int32(_C)
        for l in range(_NUM_LEVELS):
            pltpu.sync_copy(
                scr_hbm.at[pl.ds(jnp.int32(l * 2 * _N)
                                 + pbase * jnp.int32(2), 2 * _C)],
                tr_v.at[jnp.int32(l)])

        def grp(g, c2):
            s32 = g * jnp.int32(32)
            for l in range(_NUM_LEVELS):
                v_lo = tr_v[jnp.int32(l), pl.ds(s32, 16)]
                v_hi = tr_v[jnp.int32(l), pl.ds(s32 + jnp.int32(16), 16)]
                pat = patL + (g * jnp.int32(512) + jnp.int32(2 * l))
                plsc.store_scatter(out_v, [pat], v_lo)
                plsc.store_scatter(out_v, [pat + jnp.int32(256)], v_hi)
            return c2

        lax.fori_loop(jnp.int32(0), jnp.int32(_G), grp, jnp.int32(0))
        pltpu.sync_copy(out_v,
                        out_hbm.at[pl.ds(pbase * jnp.int32(32), _C * 32)])
        return carry

    lax.fori_loop(jnp.int32(0), jnp.int32(_NCHUNK), tr_chunk, jnp.int32(0))


def kernel(x, tables):
    mesh = plsc.VectorSubcoreMesh(
        core_axis_name="c", subcore_axis_name="s",
        num_cores=_NC, num_subcores=_NS)
    k = pl.kernel(
        _body,
        out_type=(
            jax.ShapeDtypeStruct((_NUM_LEVELS * 2 * _N,), jnp.float32),
            jax.ShapeDtypeStruct((_N * _NUM_LEVELS * _F,), jnp.float32),
        ),
        mesh=mesh,
        scratch_types=[
            pltpu.VMEM_SHARED((_LW,), jnp.float32),
            pltpu.VMEM((3, _C), jnp.float32),
            pltpu.VMEM((3, _C), jnp.float32),
            pltpu.VMEM((8, _C), jnp.int32),
            pltpu.VMEM((8, _C), jnp.int32),
            pltpu.VMEM((8, _C), jnp.int32),
            pltpu.VMEM((8, _C), jnp.int32),
            pltpu.VMEM((8, _C), jnp.float32),
            pltpu.VMEM((8, _C), jnp.float32),
            pltpu.VMEM((8, _C), jnp.float32),
            pltpu.VMEM((8, _C), jnp.float32),
            pltpu.VMEM((2 * _C,), jnp.float32),
            pltpu.VMEM((_NUM_LEVELS * 16,), jnp.float32),
            pltpu.VMEM((_NUM_LEVELS, 2 * _C), jnp.float32),
            pltpu.VMEM((_C * 32,), jnp.float32),
            pltpu.SemaphoreType.DMA,
            pltpu.SemaphoreType.DMA,
        ],
        compiler_params=pltpu.CompilerParams(
            needs_layout_passes=False, use_tc_tiling_on_sc=False),
    )
    xt = x.astype(jnp.float32).T.reshape(3 * _N)
    tab = tables.astype(jnp.float32).reshape(_NUM_LEVELS * _LW)
    resb = jnp.broadcast_to(
        jnp.asarray(_RES, jnp.float32)[:, None],
        (_NUM_LEVELS, 16)).reshape(_NUM_LEVELS * 16)
    _, out = k(xt, tab, resb)
    return out.reshape(_N, _NUM_LEVELS, _F)


# speedup vs baseline: 15.7381x; 15.7381x over previous
import math

import jax
import jax.numpy as jnp
from jax import lax
from jax.experimental import pallas as pl
from jax.experimental.pallas import tpu as pltpu
from jax.experimental.pallas import tpu_sc as plsc

_NUM_LEVELS = 16
_F = 2
_T = 2 ** 19
_MASK = _T - 1
_LW = _T * _F
_BASE_RES = 16
_FINEST_RES = 2048
_GROWTH = math.exp(
    (math.log(_FINEST_RES) - math.log(_BASE_RES)) / (_NUM_LEVELS - 1))
_RES = [max(1, int(round(_BASE_RES * _GROWTH ** l)))
        for l in range(_NUM_LEVELS)]
_N = 262144
_NC, _NS = 2, 16
_NW = _NC * _NS
_PW = _N // _NW
_C = 128
_NCHUNK = _PW // _C
_G = _C // 16
_PA = 73856093
_PB = 19349663
_PC = 83492791


def _body(x_hbm, tab_hbm, resb_hbm, out_hbm, sh,
          x0, x1, idxE0, idxE1, idxO0, idxO1, fE0, fE1, fO0, fO1,
          res_v, resb_v, sem0, sem1):
    cid = lax.convert_element_type(lax.axis_index("c"), jnp.int32)
    sid = lax.convert_element_type(lax.axis_index("s"), jnp.int32)
    wid32 = sid * jnp.int32(_NC) + cid
    x_b = (x0, x1)
    idxE_b = (idxE0, idxE1)
    idxO_b = (idxO0, idxO1)
    fE_b = (fE0, fE1)
    fO_b = (fO0, fO1)
    sem_b = (sem0, sem1)
    wblock = wid32 * jnp.int32(_NCHUNK)
    stage_per = _LW // _NS

    def pass1(p, blk, res):
        x_v, idxE, idxO = x_b[p], idxE_b[p], idxO_b[p]
        pltpu.sync_copy(x_hbm.at[pl.ds(blk * jnp.int32(384), 384)], x_v)

        def grp(g, c2):
            s = g * jnp.int32(16)
            xs0 = x_v[pl.ds(s, 16)] * res
            xs1 = x_v[pl.ds(s + jnp.int32(128), 16)] * res
            xs2 = x_v[pl.ds(s + jnp.int32(256), 16)] * res
            i0 = xs0.astype(jnp.int32)
            i1 = xs1.astype(jnp.int32)
            i2 = xs2.astype(jnp.int32)
            a0 = i0 * jnp.int32(_PA)
            a1 = a0 + jnp.int32(_PA)
            b0 = i1 * jnp.int32(_PB)
            b1 = b0 + jnp.int32(_PB)
            c0 = i2 * jnp.int32(_PC)
            c1 = c0 + jnp.int32(_PC)
            corner = 0
            for aa in (a0, a1):
                for bb in (b0, b1):
                    for cc in (c0, c1):
                        h = (aa ^ bb ^ cc) & jnp.int32(_MASK)
                        lo = h & jnp.int32(127)
                        e = lax.shift_left(h - lo, jnp.int32(1)) + lo
                        idxE[jnp.int32(corner), pl.ds(s, 16)] = e
                        idxO[jnp.int32(corner), pl.ds(s, 16)] = \
                            e + jnp.int32(128)
                        corner += 1
            return c2

        lax.fori_loop(jnp.int32(0), jnp.int32(_G), grp, jnp.int32(0))
        for c in range(8):
            pltpu.async_copy(sh.at[idxE.at[jnp.int32(c)]],
                             fE_b[p].at[jnp.int32(c)], sem_b[p])
            pltpu.async_copy(sh.at[idxO.at[jnp.int32(c)]],
                             fO_b[p].at[jnp.int32(c)], sem_b[p])

    def drain(p):
        dummy = tab_hbm.at[pl.ds(0, _C)]
        for c in range(8):
            pltpu.make_async_copy(dummy, fE_b[p].at[jnp.int32(c)],
                                  sem_b[p]).wait()
            pltpu.make_async_copy(dummy, fO_b[p].at[jnp.int32(c)],
                                  sem_b[p]).wait()

    def combine(p, blk, res, l):
        x_v, fE, fO = x_b[p], fE_b[p], fO_b[p]

        def grp(g, c2):
            s = g * jnp.int32(16)
            xs0 = x_v[pl.ds(s, 16)] * res
            xs1 = x_v[pl.ds(s + jnp.int32(128), 16)] * res
            xs2 = x_v[pl.ds(s + jnp.int32(256), 16)] * res
            w0 = xs0 - xs0.astype(jnp.int32).astype(jnp.float32)
            w1 = xs1 - xs1.astype(jnp.int32).astype(jnp.float32)
            w2 = xs2 - xs2.astype(jnp.int32).astype(jnp.float32)
            u0 = 1.0 - w0
            u1 = 1.0 - w1
            u2 = 1.0 - w2
            p00 = u0 * u1
            p01 = u0 * w1
            p10 = w0 * u1
            p11 = w0 * w1
            wts = (p00 * u2, p00 * w2, p01 * u2, p01 * w2,
                   p10 * u2, p10 * w2, p11 * u2, p11 * w2)
            acc0 = None
            acc1 = None
            for c in range(8):
                f0 = fE[jnp.int32(c), pl.ds(s, 16)]
                f1 = fO[jnp.int32(c), pl.ds(s, 16)]
                t0 = wts[c] * f0
                t1 = wts[c] * f1
                acc0 = t0 if acc0 is None else acc0 + t0
                acc1 = t1 if acc1 is None else acc1 + t1
            res_v[pl.ds(s, 16)] = acc0
            res_v[pl.ds(s + jnp.int32(128), 16)] = acc1
            return c2

        lax.fori_loop(jnp.int32(0), jnp.int32(_G), grp, jnp.int32(0))
        pltpu.sync_copy(
            res_v,
            out_hbm.at[pl.ds(l * jnp.int32(2 * _N) + blk * jnp.int32(256),
                             256)])

    pltpu.sync_copy(resb_hbm, resb_v)

    def level_body(l, carry0):
        res = resb_v[pl.ds(l * jnp.int32(16), 16)]
        plsc.subcore_barrier()
        pltpu.sync_copy(
            tab_hbm.at[pl.ds(l * jnp.int32(_LW)
                             + sid * jnp.int32(stage_per), stage_per)],
            sh.at[pl.ds(sid * jnp.int32(stage_per), stage_per)])
        plsc.subcore_barrier()

        pass1(0, wblock, res)

        def pair_body(j, carry):
            blk0 = wblock + j * jnp.int32(2)
            pass1(1, blk0 + jnp.int32(1), res)
            drain(0)
            combine(0, blk0, res, l)

            @pl.when(j < jnp.int32(_NCHUNK // 2 - 1))
            def _():
                pass1(0, blk0 + jnp.int32(2), res)

            drain(1)
            combine(1, blk0 + jnp.int32(1), res, l)
            return carry

        lax.fori_loop(jnp.int32(0), jnp.int32(_NCHUNK // 2), pair_body,
                      jnp.int32(0))
        return carry0

    lax.fori_loop(jnp.int32(0), jnp.int32(_NUM_LEVELS), level_body,
                  jnp.int32(0))


def kernel(x, tables):
    mesh = plsc.VectorSubcoreMesh(
        core_axis_name="c", subcore_axis_name="s",
        num_cores=_NC, num_subcores=_NS)
    k = pl.kernel(
        _body,
        out_type=jax.ShapeDtypeStruct((_NUM_LEVELS * _N * _F,), jnp.float32),
        mesh=mesh,
        scratch_types=[
            pltpu.VMEM_SHARED((_LW,), jnp.float32),
            pltpu.VMEM((3 * _C,), jnp.float32),
            pltpu.VMEM((3 * _C,), jnp.float32),
            pltpu.VMEM((8, _C), jnp.int32),
            pltpu.VMEM((8, _C), jnp.int32),
            pltpu.VMEM((8, _C), jnp.int32),
            pltpu.VMEM((8, _C), jnp.int32),
            pltpu.VMEM((8, _C), jnp.float32),
            pltpu.VMEM((8, _C), jnp.float32),
            pltpu.VMEM((8, _C), jnp.float32),
            pltpu.VMEM((8, _C), jnp.float32),
            pltpu.VMEM((2 * _C,), jnp.float32),
            pltpu.VMEM((_NUM_LEVELS * 16,), jnp.float32),
            pltpu.SemaphoreType.DMA,
            pltpu.SemaphoreType.DMA,
        ],
        compiler_params=pltpu.CompilerParams(
            needs_layout_passes=False, use_tc_tiling_on_sc=False),
    )
    xf = x.astype(jnp.float32)
    xin = xf.reshape(_N // _C, _C, 3).transpose(0, 2, 1).reshape(3 * _N)
    tab = tables.astype(jnp.float32) \
        .reshape(_NUM_LEVELS, _T // _C, _C, _F) \
        .transpose(0, 1, 3, 2).reshape(_NUM_LEVELS * _LW)
    resb = jnp.broadcast_to(
        jnp.asarray(_RES, jnp.float32)[:, None],
        (_NUM_LEVELS, 16)).reshape(_NUM_LEVELS * 16)
    out = k(xin, tab, resb)
    return out.reshape(_NUM_LEVELS, _N // _C, _F, _C) \
        .transpose(1, 3, 0, 2).reshape(_N, _NUM_LEVELS, _F)
